# Initial kernel scaffold; baseline (speedup 1.0000x reference)
import functools

import jax
import jax.numpy as jnp
from jax import lax
from jax.experimental import pallas as pl
from jax.experimental.pallas import tpu as pltpu
from jax.experimental.pallas import tpu_sc as plsc

NC = 2
NS = 16
L = 16
NW = NC * NS

GROWS = 4
C = GROWS * 128


def _input_stage(x_pad, W1, b1_2d, beta_2d, n_real, npad):
    def body(x_ref, w_ref, b_ref, beta_ref, hn_ref, hnb_ref, nrm_ref):
        h = jnp.dot(x_ref[...], w_ref[...], preferred_element_type=jnp.float32)
        h = jnp.maximum(h + b_ref[...], 0.0)
        rows = lax.broadcasted_iota(jnp.int32, h.shape, 0)
        h = jnp.where(rows < n_real, h, 0.0)
        nrm = jnp.sqrt(jnp.sum(h * h, axis=1, keepdims=True))
        hn = h / jnp.maximum(nrm, 1e-12)
        hn_ref[...] = hn
        hnb_ref[...] = hn * beta_ref[0, 0]
        nrm_ref[...] = nrm

    hid = W1.shape[1]
    return pl.pallas_call(
        body,
        out_shape=[
            jax.ShapeDtypeStruct((npad, hid), jnp.float32),
            jax.ShapeDtypeStruct((npad, hid), jnp.float32),
            jax.ShapeDtypeStruct((npad, 1), jnp.float32),
        ],
    )(x_pad, W1, b1_2d, beta_2d)


def _combine_stage(acc, den, beta_2d):
    def body(acc_ref, den_ref, beta_ref, hn_ref, hnb_ref, nrm_ref):
        A = acc_ref[0] + acc_ref[1]
        d = den_ref[0] + den_ref[1]
        nA = jnp.sqrt(jnp.sum(A * A, axis=1, keepdims=True))
        hn = A / jnp.maximum(nA, 1e-12)
        hn_ref[...] = hn
        hnb_ref[...] = hn * beta_ref[0, 0]
        nrm_ref[...] = nA / jnp.maximum(d[:, None], 1e-30)

    npad, hid = acc.shape[1], acc.shape[2]
    return pl.pallas_call(
        body,
        out_shape=[
            jax.ShapeDtypeStruct((npad, hid), jnp.float32),
            jax.ShapeDtypeStruct((npad, hid), jnp.float32),
            jax.ShapeDtypeStruct((npad, 1), jnp.float32),
        ],
    )(acc, den, beta_2d)


def _output_stage(acc, den, W2, b2_2d):
    def body(acc_ref, den_ref, w_ref, b_ref, out_ref):
        A = acc_ref[0] + acc_ref[1]
        d = den_ref[0] + den_ref[1]
        h = A / jnp.maximum(d[:, None], 1e-30)
        logits = jnp.dot(h, w_ref[...], preferred_element_type=jnp.float32)
        logits = logits + b_ref[...]
        m = jnp.max(logits, axis=1, keepdims=True)
        e = jnp.exp(logits - m)
        out_ref[...] = e / jnp.sum(e, axis=1, keepdims=True)

    npad = acc.shape[1]
    ncls = W2.shape[1]
    return pl.pallas_call(
        body,
        out_shape=jax.ShapeDtypeStruct((npad, ncls), jnp.float32),
    )(acc, den, W2, b2_2d)


def _agnn_sparse_pass(hn, hnb, nrm, srcm, dstm):
    npad, hid = hn.shape
    erows = srcm.shape[0]
    rows_per_tile = erows // NW
    chunks = rows_per_tile // GROWS
    stripe = npad // NS

    mesh = plsc.VectorSubcoreMesh(core_axis_name="c", subcore_axis_name="s")

    @functools.partial(
        pl.kernel,
        out_type=[
            jax.ShapeDtypeStruct((NC, npad, hid), jnp.float32),
            jax.ShapeDtypeStruct((NC, npad), jnp.float32),
        ],
        mesh=mesh,
        scratch_types=[
            pltpu.VMEM((npad,), jnp.float32),
            pltpu.VMEM((GROWS, 128), jnp.int32),
            pltpu.VMEM((GROWS, 128), jnp.int32),
            pltpu.VMEM((C, hid), jnp.float32),
            pltpu.VMEM((C, hid), jnp.float32),
            pltpu.VMEM((C, hid), jnp.float32),
            pltpu.VMEM((C,), jnp.float32),
            pltpu.VMEM((npad // NS, hid), jnp.float32),
            pltpu.VMEM((npad // NS,), jnp.float32),
            pltpu.VMEM_SHARED((npad, hid), jnp.float32),
            pltpu.VMEM_SHARED((npad,), jnp.float32),
            pltpu.SemaphoreType.DMA,
            pltpu.SemaphoreType.DMA,
        ],
    )
    def k(hn_hbm, hnb_hbm, nrm_hbm, src_hbm, dst_hbm, acc_hbm, den_hbm,
          nrm_v, src_v, dst_v, hs_v, hd_v, msg_v, w_v, z_v, zd_v,
          acc_sh, den_sh, sem1, sem2):
        cid = lax.axis_index("c")
        sid = lax.axis_index("s")
        wid = cid * NS + sid

        pltpu.sync_copy(nrm_hbm, nrm_v)

        @pl.loop(0, stripe)
        def _(r):
            z_v[r, :] = jnp.zeros((L,), jnp.float32)

        @pl.loop(0, stripe, step=L)
        def _(i):
            zd_v[pl.ds(i, L)] = jnp.zeros((L,), jnp.float32)

        base_row = sid * stripe
        pltpu.sync_copy(z_v, acc_sh.at[pl.ds(base_row, stripe)])
        pltpu.sync_copy(zd_v, den_sh.at[pl.ds(base_row, stripe)])
        plsc.subcore_barrier()

        my_row0 = wid * rows_per_tile
        lane = lax.iota(jnp.int32, L)

        @pl.loop(0, chunks)
        def _(t):
            row0 = my_row0 + t * GROWS
            pltpu.sync_copy(src_hbm.at[pl.ds(row0, GROWS)], src_v)
            pltpu.sync_copy(dst_hbm.at[pl.ds(row0, GROWS)], dst_v)
            cps = []
            for g in range(GROWS):
                cps.append(pltpu.async_copy(
                    hn_hbm.at[src_v.at[g]],
                    hs_v.at[pl.ds(g * 128, 128)], sem1))
                cps.append(pltpu.async_copy(
                    hnb_hbm.at[dst_v.at[g]],
                    hd_v.at[pl.ds(g * 128, 128)], sem2))
            for cp in cps:
                cp.wait()

            @pl.loop(0, C // L)
            def _(q):
                r0 = q * L
                rows = r0 + lane
                acc = jnp.zeros((L,), jnp.float32)
                a_list = []
                for f in range(hid):
                    col = jnp.full((L,), f, jnp.int32)
                    a = plsc.load_gather(hs_v, [rows, col])
                    b = plsc.load_gather(hd_v, [rows, col])
                    a_list.append(a)
                    acc = acc + a * b
                w16 = jnp.exp(acc)
                g_idx = q // (128 // L)
                c0 = (q % (128 // L)) * L
                src16 = src_v[g_idx, pl.ds(c0, L)]
                ns16 = plsc.load_gather(nrm_v, [src16])
                v16 = w16 * ns16
                w_v[pl.ds(r0, L)] = w16
                for f in range(hid):
                    col = jnp.full((L,), f, jnp.int32)
                    plsc.store_scatter(msg_v, [rows, col], v16 * a_list[f])

            for g in range(GROWS):
                pltpu.sync_copy(msg_v.at[pl.ds(g * 128, 128)],
                                acc_sh.at[dst_v.at[g]], add=True)
                pltpu.sync_copy(w_v.at[pl.ds(g * 128, 128)],
                                den_sh.at[dst_v.at[g]], add=True)

        plsc.subcore_barrier()
        pltpu.sync_copy(acc_sh.at[pl.ds(base_row, stripe)],
                        acc_hbm.at[cid, pl.ds(base_row, stripe)])
        pltpu.sync_copy(den_sh.at[pl.ds(base_row, stripe)],
                        den_hbm.at[cid, pl.ds(base_row, stripe)])

    return k(hn, hnb, nrm, srcm, dstm)


def kernel(x, edge_index, W1, b1, beta1, beta2, beta3, W2, b2):
    n, nfeat = x.shape
    e = edge_index.shape[1]
    hid = W1.shape[1]

    npad = ((n + 16) + NW * L - 1) // (NW * L) * (NW * L)
    etot = e + n
    epad = (etot + NW * C - 1) // (NW * C) * (NW * C)
    erows = epad // 128

    loop_idx = jnp.arange(n, dtype=jnp.int32)
    src = jnp.concatenate([edge_index[0].astype(jnp.int32), loop_idx,
                           jnp.zeros((epad - etot,), jnp.int32)])
    dst = jnp.concatenate([edge_index[1].astype(jnp.int32), loop_idx,
                           jnp.full((epad - etot,), n, jnp.int32)])
    srcm = src.reshape(erows, 128)
    dstm = dst.reshape(erows, 128)

    x_pad = jnp.pad(x, ((0, npad - n), (0, 0)))
    b1_2d = b1.reshape(1, hid)
    beta1_2d = jnp.reshape(beta1, (1, 1)).astype(jnp.float32)
    beta2_2d = jnp.reshape(beta2, (1, 1)).astype(jnp.float32)
    beta3_2d = jnp.reshape(beta3, (1, 1)).astype(jnp.float32)

    hn, hnb, nrm2 = _input_stage(x_pad, W1, b1_2d, beta1_2d, n, npad)
    nrm = nrm2.reshape(npad)

    acc, den = _agnn_sparse_pass(hn, hnb, nrm, srcm, dstm)
    hn, hnb, nrm2 = _combine_stage(acc, den, beta2_2d)
    nrm = nrm2.reshape(npad)

    acc, den = _agnn_sparse_pass(hn, hnb, nrm, srcm, dstm)
    hn, hnb, nrm2 = _combine_stage(acc, den, beta3_2d)
    nrm = nrm2.reshape(npad)

    acc, den = _agnn_sparse_pass(hn, hnb, nrm, srcm, dstm)
    out = _output_stage(acc, den, W2, b2.reshape(1, -1))
    return out[:n]


# speedup vs baseline: 25.8800x; 25.8800x over previous
import dataclasses
import functools

import jax
import jax.numpy as jnp
from jax import lax
from jax.experimental import pallas as pl
from jax.experimental.pallas import tpu as pltpu
from jax.experimental.pallas import tpu_sc as plsc

NC = 2
NS = 16
L = 16
NW = NC * NS

GROWS = 4
C = GROWS * 128


def _input_stage(x_pad, W1, b1_2d, beta_2d, n_real, npad):
    def body(x_ref, w_ref, b_ref, beta_ref, hn_ref, hnb_ref, nrm_ref):
        h = jnp.dot(x_ref[...], w_ref[...], preferred_element_type=jnp.float32)
        h = jnp.maximum(h + b_ref[...], 0.0)
        rows = lax.broadcasted_iota(jnp.int32, h.shape, 0)
        h = jnp.where(rows < n_real, h, 0.0)
        nrm = jnp.sqrt(jnp.sum(h * h, axis=1, keepdims=True))
        hn = h / jnp.maximum(nrm, 1e-12)
        hn_ref[...] = hn
        hnb_ref[...] = hn * beta_ref[0, 0]
        nrm_ref[...] = nrm

    hid = W1.shape[1]
    return pl.pallas_call(
        body,
        out_shape=[
            jax.ShapeDtypeStruct((npad, hid), jnp.float32),
            jax.ShapeDtypeStruct((npad, hid), jnp.float32),
            jax.ShapeDtypeStruct((npad, 1), jnp.float32),
        ],
    )(x_pad, W1, b1_2d, beta_2d)


def _combine_stage(acc, den, beta_2d):
    def body(acc_ref, den_ref, beta_ref, hn_ref, hnb_ref, nrm_ref):
        A = acc_ref[0] + acc_ref[1]
        d = den_ref[0] + den_ref[1]
        nA = jnp.sqrt(jnp.sum(A * A, axis=1, keepdims=True))
        hn = A / jnp.maximum(nA, 1e-12)
        hn_ref[...] = hn
        hnb_ref[...] = hn * beta_ref[0, 0]
        nrm_ref[...] = nA / jnp.maximum(d, 1e-30)

    npad, hid = acc.shape[1], acc.shape[2]
    return pl.pallas_call(
        body,
        out_shape=[
            jax.ShapeDtypeStruct((npad, hid), jnp.float32),
            jax.ShapeDtypeStruct((npad, hid), jnp.float32),
            jax.ShapeDtypeStruct((npad, 1), jnp.float32),
        ],
    )(acc, den.reshape(NC, npad, 1), beta_2d)


def _output_stage(acc, den, W2, b2_2d):
    def body(acc_ref, den_ref, w_ref, b_ref, out_ref):
        A = acc_ref[0] + acc_ref[1]
        d = den_ref[0] + den_ref[1]
        h = A / jnp.maximum(d, 1e-30)
        logits = jnp.dot(h, w_ref[...], preferred_element_type=jnp.float32)
        logits = logits + b_ref[...]
        m = jnp.max(logits, axis=1, keepdims=True)
        e = jnp.exp(logits - m)
        out_ref[...] = e / jnp.sum(e, axis=1, keepdims=True)

    npad = acc.shape[1]
    ncls = W2.shape[1]
    return pl.pallas_call(
        body,
        out_shape=jax.ShapeDtypeStruct((npad, ncls), jnp.float32),
    )(acc, den.reshape(NC, npad, 1), W2, b2_2d)


def _agnn_sparse_pass(hn, hnb, nrm, srcm, dstm):
    npad, hid = hn.shape
    erows = srcm.shape[0]
    rows_per_tile = erows // NW
    chunks = rows_per_tile // GROWS
    stripe = npad // NS

    mesh = plsc.VectorSubcoreMesh(core_axis_name="c", subcore_axis_name="s")

    cp = pltpu.CompilerParams()
    for fld, val in (("needs_layout_passes", False),
                     ("use_tc_tiling_on_sc", False)):
        if fld in pltpu.CompilerParams.__dataclass_fields__:
            cp = dataclasses.replace(cp, **{fld: val})

    @functools.partial(
        pl.kernel,
        compiler_params=cp,
        out_type=[
            jax.ShapeDtypeStruct((NC, npad, hid), jnp.float32),
            jax.ShapeDtypeStruct((NC, npad), jnp.float32),
        ],
        mesh=mesh,
        scratch_types=[
            pltpu.VMEM((npad,), jnp.float32),
            pltpu.VMEM((GROWS, 128), jnp.int32),
            pltpu.VMEM((GROWS, 128), jnp.int32),
            pltpu.VMEM((C, hid), jnp.float32),
            pltpu.VMEM((C, hid), jnp.float32),
            pltpu.VMEM((C, hid), jnp.float32),
            pltpu.VMEM((C,), jnp.float32),
            pltpu.VMEM((npad // NS, hid), jnp.float32),
            pltpu.VMEM((npad // NS,), jnp.float32),
            pltpu.VMEM_SHARED((npad, hid), jnp.float32),
            pltpu.VMEM_SHARED((npad,), jnp.float32),
            pltpu.SemaphoreType.DMA,
            pltpu.SemaphoreType.DMA,
        ],
    )
    def k(hn_hbm, hnb_hbm, nrm_hbm, src_hbm, dst_hbm, acc_hbm, den_hbm,
          nrm_v, src_v, dst_v, hs_v, hd_v, msg_v, w_v, z_v, zd_v,
          acc_sh, den_sh, sem1, sem2):
        cid = lax.axis_index("c")
        sid = lax.axis_index("s")
        wid = cid * NS + sid

        pltpu.sync_copy(nrm_hbm, nrm_v)

        @pl.loop(0, stripe)
        def _(r):
            z_v[r, :] = jnp.zeros((L,), jnp.float32)

        @pl.loop(0, stripe, step=L)
        def _(i):
            zd_v[pl.ds(i, L)] = jnp.zeros((L,), jnp.float32)

        base_row = sid * stripe
        pltpu.sync_copy(z_v, acc_sh.at[pl.ds(base_row, stripe)])
        pltpu.sync_copy(zd_v, den_sh.at[pl.ds(base_row, stripe)])
        plsc.subcore_barrier()

        my_row0 = wid * rows_per_tile
        lane = lax.iota(jnp.int32, L)

        @pl.loop(0, chunks)
        def _(t):
            row0 = my_row0 + t * GROWS
            pltpu.sync_copy(src_hbm.at[pl.ds(row0, GROWS)], src_v)
            pltpu.sync_copy(dst_hbm.at[pl.ds(row0, GROWS)], dst_v)
            cps = []
            for g in range(GROWS):
                cps.append(pltpu.async_copy(
                    hn_hbm.at[src_v.at[g]],
                    hs_v.at[pl.ds(g * 128, 128)], sem1))
                cps.append(pltpu.async_copy(
                    hnb_hbm.at[dst_v.at[g]],
                    hd_v.at[pl.ds(g * 128, 128)], sem2))
            for cp in cps:
                cp.wait()

            @pl.loop(0, C // L)
            def _(q):
                r0 = q * L
                rows = r0 + lane
                acc = jnp.zeros((L,), jnp.float32)
                a_list = []
                for f in range(hid):
                    col = jnp.full((L,), f, jnp.int32)
                    a = plsc.load_gather(hs_v, [rows, col])
                    b = plsc.load_gather(hd_v, [rows, col])
                    a_list.append(a)
                    acc = acc + a * b
                w16 = jnp.exp(acc)
                g_idx = q // (128 // L)
                c0 = (q % (128 // L)) * L
                src16 = src_v[g_idx, pl.ds(c0, L)]
                ns16 = plsc.load_gather(nrm_v, [src16])
                v16 = w16 * ns16
                w_v[pl.ds(r0, L)] = w16
                for f in range(hid):
                    col = jnp.full((L,), f, jnp.int32)
                    plsc.store_scatter(msg_v, [rows, col], v16 * a_list[f])

            for g in range(GROWS):
                pltpu.sync_copy(msg_v.at[pl.ds(g * 128, 128)],
                                acc_sh.at[dst_v.at[g]], add=True)
                pltpu.sync_copy(w_v.at[pl.ds(g * 128, 128)],
                                den_sh.at[dst_v.at[g]], add=True)

        plsc.subcore_barrier()
        pltpu.sync_copy(acc_sh.at[pl.ds(base_row, stripe)],
                        acc_hbm.at[cid, pl.ds(base_row, stripe)])
        pltpu.sync_copy(den_sh.at[pl.ds(base_row, stripe)],
                        den_hbm.at[cid, pl.ds(base_row, stripe)])

    return k(hn, hnb, nrm, srcm, dstm)


def kernel(x, edge_index, W1, b1, beta1, beta2, beta3, W2, b2):
    n, nfeat = x.shape
    e = edge_index.shape[1]
    hid = W1.shape[1]

    npad = ((n + 16) + NW * L - 1) // (NW * L) * (NW * L)
    etot = e + n
    epad = (etot + NW * C - 1) // (NW * C) * (NW * C)
    erows = epad // 128

    loop_idx = jnp.arange(n, dtype=jnp.int32)
    src = jnp.concatenate([edge_index[0].astype(jnp.int32), loop_idx,
                           jnp.zeros((epad - etot,), jnp.int32)])
    dst = jnp.concatenate([edge_index[1].astype(jnp.int32), loop_idx,
                           jnp.full((epad - etot,), n, jnp.int32)])
    srcm = src.reshape(erows, 128)
    dstm = dst.reshape(erows, 128)

    x_pad = jnp.pad(x, ((0, npad - n), (0, 0)))
    b1_2d = b1.reshape(1, hid)
    beta1_2d = jnp.reshape(beta1, (1, 1)).astype(jnp.float32)
    beta2_2d = jnp.reshape(beta2, (1, 1)).astype(jnp.float32)
    beta3_2d = jnp.reshape(beta3, (1, 1)).astype(jnp.float32)

    hn, hnb, nrm2 = _input_stage(x_pad, W1, b1_2d, beta1_2d, n, npad)
    nrm = nrm2.reshape(npad)

    acc, den = _agnn_sparse_pass(hn, hnb, nrm, srcm, dstm)
    hn, hnb, nrm2 = _combine_stage(acc, den, beta2_2d)
    nrm = nrm2.reshape(npad)

    acc, den = _agnn_sparse_pass(hn, hnb, nrm, srcm, dstm)
    hn, hnb, nrm2 = _combine_stage(acc, den, beta3_2d)
    nrm = nrm2.reshape(npad)

    acc, den = _agnn_sparse_pass(hn, hnb, nrm, srcm, dstm)
    out = _output_stage(acc, den, W2, b2.reshape(1, -1))
    return out[:n]


# speedup vs baseline: 44.3752x; 1.7147x over previous
import dataclasses
import functools

import jax
import jax.numpy as jnp
from jax import lax
from jax.experimental import pallas as pl
from jax.experimental.pallas import tpu as pltpu
from jax.experimental.pallas import tpu_sc as plsc

NC = 2
NS = 16
L = 16
NW = NC * NS

GROWS = 4
C = GROWS * 128


def _input_stage(x_pad, W1, b1_2d, beta_2d, n_real, npad):
    def body(x_ref, w_ref, b_ref, beta_ref, hn_ref, hnb_ref, nrm_ref):
        h = jnp.dot(x_ref[...], w_ref[...], preferred_element_type=jnp.float32)
        h = jnp.maximum(h + b_ref[...], 0.0)
        rows = lax.broadcasted_iota(jnp.int32, h.shape, 0)
        h = jnp.where(rows < n_real, h, 0.0)
        nrm = jnp.sqrt(jnp.sum(h * h, axis=1, keepdims=True))
        hn = h / jnp.maximum(nrm, 1e-12)
        hn_ref[...] = hn
        hnb_ref[...] = hn * beta_ref[0, 0]
        nrm_ref[...] = nrm

    hid = W1.shape[1]
    return pl.pallas_call(
        body,
        out_shape=[
            jax.ShapeDtypeStruct((npad, hid), jnp.float32),
            jax.ShapeDtypeStruct((npad, hid), jnp.float32),
            jax.ShapeDtypeStruct((npad, 1), jnp.float32),
        ],
    )(x_pad, W1, b1_2d, beta_2d)


def _combine_stage(acc, den, beta_2d):
    def body(acc_ref, den_ref, beta_ref, hn_ref, hnb_ref, nrm_ref):
        A = acc_ref[0] + acc_ref[1]
        d = den_ref[0] + den_ref[1]
        nA = jnp.sqrt(jnp.sum(A * A, axis=1, keepdims=True))
        hn = A / jnp.maximum(nA, 1e-12)
        hn_ref[...] = hn
        hnb_ref[...] = hn * beta_ref[0, 0]
        nrm_ref[...] = nA / jnp.maximum(d, 1e-30)

    npad, hid = acc.shape[1], acc.shape[2]
    return pl.pallas_call(
        body,
        out_shape=[
            jax.ShapeDtypeStruct((npad, hid), jnp.float32),
            jax.ShapeDtypeStruct((npad, hid), jnp.float32),
            jax.ShapeDtypeStruct((npad, 1), jnp.float32),
        ],
    )(acc, den.reshape(NC, npad, 1), beta_2d)


def _output_stage(acc, den, W2, b2_2d):
    def body(acc_ref, den_ref, w_ref, b_ref, out_ref):
        A = acc_ref[0] + acc_ref[1]
        d = den_ref[0] + den_ref[1]
        h = A / jnp.maximum(d, 1e-30)
        logits = jnp.dot(h, w_ref[...], preferred_element_type=jnp.float32)
        logits = logits + b_ref[...]
        m = jnp.max(logits, axis=1, keepdims=True)
        e = jnp.exp(logits - m)
        out_ref[...] = e / jnp.sum(e, axis=1, keepdims=True)

    npad = acc.shape[1]
    ncls = W2.shape[1]
    return pl.pallas_call(
        body,
        out_shape=jax.ShapeDtypeStruct((npad, ncls), jnp.float32),
    )(acc, den.reshape(NC, npad, 1), W2, b2_2d)


def _agnn_sparse_pass(hn, hnb, nrm, srcm, dstm):
    npad, hid = hn.shape
    erows = srcm.shape[0]
    rows_per_tile = erows // NW
    chunks = rows_per_tile // GROWS
    pairs = chunks // 2
    stripe = npad // NS

    mesh = plsc.VectorSubcoreMesh(core_axis_name="c", subcore_axis_name="s")

    cp = pltpu.CompilerParams()
    for fld, val in (("needs_layout_passes", False),
                     ("use_tc_tiling_on_sc", False)):
        if fld in pltpu.CompilerParams.__dataclass_fields__:
            cp = dataclasses.replace(cp, **{fld: val})

    @functools.partial(
        pl.kernel,
        compiler_params=cp,
        out_type=[
            jax.ShapeDtypeStruct((NC, npad, hid), jnp.float32),
            jax.ShapeDtypeStruct((NC, npad), jnp.float32),
        ],
        mesh=mesh,
        scratch_types=[
            pltpu.VMEM((npad,), jnp.float32),
            pltpu.VMEM((rows_per_tile, 128), jnp.int32),
            pltpu.VMEM((rows_per_tile, 128), jnp.int32),
            pltpu.VMEM((C, hid), jnp.float32),
            pltpu.VMEM((C, hid), jnp.float32),
            pltpu.VMEM((C, hid), jnp.float32),
            pltpu.VMEM((C,), jnp.float32),
            pltpu.VMEM((C, hid), jnp.float32),
            pltpu.VMEM((C, hid), jnp.float32),
            pltpu.VMEM((C, hid), jnp.float32),
            pltpu.VMEM((C,), jnp.float32),
            pltpu.VMEM((npad // NS, hid), jnp.float32),
            pltpu.VMEM((npad // NS,), jnp.float32),
            pltpu.VMEM_SHARED((npad, hid), jnp.float32),
            pltpu.VMEM_SHARED((npad,), jnp.float32),
            pltpu.SemaphoreType.DMA,
            pltpu.SemaphoreType.DMA,
            pltpu.SemaphoreType.DMA,
            pltpu.SemaphoreType.DMA,
        ],
    )
    def k(hn_hbm, hnb_hbm, nrm_hbm, src_hbm, dst_hbm, acc_hbm, den_hbm,
          nrm_v, src_v, dst_v, hsA, hdA, msgA, wA, hsB, hdB, msgB, wB,
          z_v, zd_v, acc_sh, den_sh, semgA, semgB, semsA, semsB):
        cid = lax.axis_index("c")
        sid = lax.axis_index("s")
        wid = cid * NS + sid
        my_row0 = wid * rows_per_tile

        pltpu.sync_copy(nrm_hbm, nrm_v)
        pltpu.sync_copy(src_hbm.at[pl.ds(my_row0, rows_per_tile)], src_v)
        pltpu.sync_copy(dst_hbm.at[pl.ds(my_row0, rows_per_tile)], dst_v)

        @pl.loop(0, stripe)
        def _(r):
            z_v[r, :] = jnp.zeros((L,), jnp.float32)

        @pl.loop(0, stripe, step=L)
        def _(i):
            zd_v[pl.ds(i, L)] = jnp.zeros((L,), jnp.float32)

        base_row = sid * stripe
        pltpu.sync_copy(z_v, acc_sh.at[pl.ds(base_row, stripe)])
        pltpu.sync_copy(zd_v, den_sh.at[pl.ds(base_row, stripe)])
        plsc.subcore_barrier()

        lane = lax.iota(jnp.int32, L)

        def issue_gathers(t, hs, hd, semg):
            for g in range(GROWS):
                pltpu.async_copy(hn_hbm.at[src_v.at[t * GROWS + g]],
                                 hs.at[pl.ds(g * 128, 128)], semg)
                pltpu.async_copy(hnb_hbm.at[dst_v.at[t * GROWS + g]],
                                 hd.at[pl.ds(g * 128, 128)], semg)

        def wait_gathers(hs, hd, semg):
            pltpu.make_async_copy(hn_hbm.at[pl.ds(0, C)], hs, semg).wait()
            pltpu.make_async_copy(hnb_hbm.at[pl.ds(0, C)], hd, semg).wait()

        def issue_scatters(t, msg, wv, sems):
            for g in range(GROWS):
                pltpu.async_copy(msg.at[pl.ds(g * 128, 128)],
                                 acc_sh.at[dst_v.at[t * GROWS + g]],
                                 sems, add=True)
                pltpu.async_copy(wv.at[pl.ds(g * 128, 128)],
                                 den_sh.at[dst_v.at[t * GROWS + g]],
                                 sems, add=True)

        def wait_scatters(msg, wv, sems):
            pltpu.make_async_copy(hn_hbm.at[pl.ds(0, C)], msg, sems).wait()
            pltpu.make_async_copy(nrm_hbm.at[pl.ds(0, C)], wv, sems).wait()

        def compute(t, hs, hd, msg, wv):
            @pl.loop(0, C // L)
            def _(q):
                r0 = q * L
                rows = r0 + lane
                acc = jnp.zeros((L,), jnp.float32)
                a_list = []
                for f in range(hid):
                    col = jnp.full((L,), f, jnp.int32)
                    a = plsc.load_gather(hs, [rows, col])
                    b = plsc.load_gather(hd, [rows, col])
                    a_list.append(a)
                    acc = acc + a * b
                w16 = jnp.exp(acc)
                lrow = t * GROWS + q // (128 // L)
                c0 = (q % (128 // L)) * L
                src16 = src_v[lrow, pl.ds(c0, L)]
                ns16 = plsc.load_gather(nrm_v, [src16])
                v16 = w16 * ns16
                wv[pl.ds(r0, L)] = w16
                for f in range(hid):
                    col = jnp.full((L,), f, jnp.int32)
                    plsc.store_scatter(msg, [rows, col], v16 * a_list[f])

        issue_gathers(0, hsA, hdA, semgA)
        issue_gathers(1, hsB, hdB, semgB)

        @pl.loop(0, pairs)
        def _(tt):
            t0 = 2 * tt
            t1 = t0 + 1

            wait_gathers(hsA, hdA, semgA)

            @pl.when(tt > 0)
            def _():
                wait_scatters(msgA, wA, semsA)

            compute(t0, hsA, hdA, msgA, wA)
            issue_scatters(t0, msgA, wA, semsA)

            @pl.when(tt < pairs - 1)
            def _():
                issue_gathers(t0 + 2, hsA, hdA, semgA)

            wait_gathers(hsB, hdB, semgB)

            @pl.when(tt > 0)
            def _():
                wait_scatters(msgB, wB, semsB)

            compute(t1, hsB, hdB, msgB, wB)
            issue_scatters(t1, msgB, wB, semsB)

            @pl.when(tt < pairs - 1)
            def _():
                issue_gathers(t1 + 2, hsB, hdB, semgB)

        wait_scatters(msgA, wA, semsA)
        wait_scatters(msgB, wB, semsB)

        plsc.subcore_barrier()
        pltpu.sync_copy(acc_sh.at[pl.ds(base_row, stripe)],
                        acc_hbm.at[cid, pl.ds(base_row, stripe)])
        pltpu.sync_copy(den_sh.at[pl.ds(base_row, stripe)],
                        den_hbm.at[cid, pl.ds(base_row, stripe)])

    return k(hn, hnb, nrm, srcm, dstm)


def kernel(x, edge_index, W1, b1, beta1, beta2, beta3, W2, b2):
    n, nfeat = x.shape
    e = edge_index.shape[1]
    hid = W1.shape[1]

    npad = ((n + 16) + NW * L - 1) // (NW * L) * (NW * L)
    etot = e + n
    epad = (etot + 2 * NW * C - 1) // (2 * NW * C) * (2 * NW * C)
    erows = epad // 128

    loop_idx = jnp.arange(n, dtype=jnp.int32)
    pad_idx = jnp.arange(epad - etot, dtype=jnp.int32)
    src = jnp.concatenate([edge_index[0].astype(jnp.int32), loop_idx,
                           pad_idx % n])
    dst = jnp.concatenate([edge_index[1].astype(jnp.int32), loop_idx,
                           n + pad_idx % (npad - n)])
    srcm = src.reshape(erows, 128)
    dstm = dst.reshape(erows, 128)

    x_pad = jnp.pad(x, ((0, npad - n), (0, 0)))
    b1_2d = b1.reshape(1, hid)
    beta1_2d = jnp.reshape(beta1, (1, 1)).astype(jnp.float32)
    beta2_2d = jnp.reshape(beta2, (1, 1)).astype(jnp.float32)
    beta3_2d = jnp.reshape(beta3, (1, 1)).astype(jnp.float32)

    hn, hnb, nrm2 = _input_stage(x_pad, W1, b1_2d, beta1_2d, n, npad)
    nrm = nrm2.reshape(npad)

    acc, den = _agnn_sparse_pass(hn, hnb, nrm, srcm, dstm)
    hn, hnb, nrm2 = _combine_stage(acc, den, beta2_2d)
    nrm = nrm2.reshape(npad)

    acc, den = _agnn_sparse_pass(hn, hnb, nrm, srcm, dstm)
    hn, hnb, nrm2 = _combine_stage(acc, den, beta3_2d)
    nrm = nrm2.reshape(npad)

    acc, den = _agnn_sparse_pass(hn, hnb, nrm, srcm, dstm)
    out = _output_stage(acc, den, W2, b2.reshape(1, -1))
    return out[:n]


# speedup vs baseline: 45.8904x; 1.0341x over previous
import dataclasses
import functools

import jax
import jax.numpy as jnp
from jax import lax
from jax.experimental import pallas as pl
from jax.experimental.pallas import tpu as pltpu
from jax.experimental.pallas import tpu_sc as plsc

NC = 2
NS = 16
L = 16
NW = NC * NS

GROWS = 4
C = GROWS * 128


def _input_stage(x_pad, W1, b1_2d, beta_2d, n_real, npad):
    def body(x_ref, w_ref, b_ref, beta_ref, hn_ref, nrm_ref, beta_row_ref):
        h = jnp.dot(x_ref[...], w_ref[...], preferred_element_type=jnp.float32)
        h = jnp.maximum(h + b_ref[...], 0.0)
        rows = lax.broadcasted_iota(jnp.int32, h.shape, 0)
        h = jnp.where(rows < n_real, h, 0.0)
        nrm = jnp.sqrt(jnp.sum(h * h, axis=1, keepdims=True))
        hn = h / jnp.maximum(nrm, 1e-12)
        hn_ref[...] = hn
        nrm_ref[...] = nrm
        beta_row_ref[...] = jnp.broadcast_to(beta_ref[0, 0], (1, 128))

    hid = W1.shape[1]
    return pl.pallas_call(
        body,
        out_shape=[
            jax.ShapeDtypeStruct((npad, hid), jnp.float32),
            jax.ShapeDtypeStruct((npad, 1), jnp.float32),
            jax.ShapeDtypeStruct((1, 128), jnp.float32),
        ],
    )(x_pad, W1, b1_2d, beta_2d)


def _combine_stage(acc, den, beta_2d):
    def body(acc_ref, den_ref, beta_ref, hn_ref, nrm_ref, beta_row_ref):
        A = acc_ref[0] + acc_ref[1]
        d = den_ref[0] + den_ref[1]
        nA = jnp.sqrt(jnp.sum(A * A, axis=1, keepdims=True))
        hn = A / jnp.maximum(nA, 1e-12)
        hn_ref[...] = hn
        nrm_ref[...] = nA / jnp.maximum(d, 1e-30)
        beta_row_ref[...] = jnp.broadcast_to(beta_ref[0, 0], (1, 128))

    npad, hid = acc.shape[1], acc.shape[2]
    return pl.pallas_call(
        body,
        out_shape=[
            jax.ShapeDtypeStruct((npad, hid), jnp.float32),
            jax.ShapeDtypeStruct((npad, 1), jnp.float32),
            jax.ShapeDtypeStruct((1, 128), jnp.float32),
        ],
    )(acc, den.reshape(NC, npad, 1), beta_2d)


def _output_stage(acc, den, W2, b2_2d):
    def body(acc_ref, den_ref, w_ref, b_ref, out_ref):
        A = acc_ref[0] + acc_ref[1]
        d = den_ref[0] + den_ref[1]
        h = A / jnp.maximum(d, 1e-30)
        logits = jnp.dot(h, w_ref[...], preferred_element_type=jnp.float32)
        logits = logits + b_ref[...]
        m = jnp.max(logits, axis=1, keepdims=True)
        e = jnp.exp(logits - m)
        out_ref[...] = e / jnp.sum(e, axis=1, keepdims=True)

    npad = acc.shape[1]
    ncls = W2.shape[1]
    return pl.pallas_call(
        body,
        out_shape=jax.ShapeDtypeStruct((npad, ncls), jnp.float32),
    )(acc, den.reshape(NC, npad, 1), W2, b2_2d)


def _agnn_sparse_pass(hn, nrm, beta_row, srcm, dstm):
    npad, hid = hn.shape
    erows = srcm.shape[0]
    rows_per_tile = erows // NW
    chunks = rows_per_tile // GROWS
    pairs = chunks // 2
    stripe = npad // NS

    mesh = plsc.VectorSubcoreMesh(core_axis_name="c", subcore_axis_name="s")

    cp = pltpu.CompilerParams()
    for fld, val in (("needs_layout_passes", False),
                     ("use_tc_tiling_on_sc", False)):
        if fld in pltpu.CompilerParams.__dataclass_fields__:
            cp = dataclasses.replace(cp, **{fld: val})

    @functools.partial(
        pl.kernel,
        compiler_params=cp,
        out_type=[
            jax.ShapeDtypeStruct((NC, npad, hid), jnp.float32),
            jax.ShapeDtypeStruct((NC, npad), jnp.float32),
        ],
        mesh=mesh,
        scratch_types=[
            pltpu.VMEM((npad,), jnp.float32),
            pltpu.VMEM((rows_per_tile, 128), jnp.int32),
            pltpu.VMEM((rows_per_tile, 128), jnp.int32),
            pltpu.VMEM((C, hid), jnp.float32),
            pltpu.VMEM((C, hid), jnp.float32),
            pltpu.VMEM((C, hid), jnp.float32),
            pltpu.VMEM((C,), jnp.float32),
            pltpu.VMEM((C, hid), jnp.float32),
            pltpu.VMEM((C, hid), jnp.float32),
            pltpu.VMEM((C, hid), jnp.float32),
            pltpu.VMEM((C,), jnp.float32),
            pltpu.VMEM((npad // NS, hid), jnp.float32),
            pltpu.VMEM((npad // NS,), jnp.float32),
            pltpu.VMEM((1, 128), jnp.float32),
            pltpu.VMEM_SHARED((npad, hid), jnp.float32),
            pltpu.VMEM_SHARED((npad, hid), jnp.float32),
            pltpu.VMEM_SHARED((npad,), jnp.float32),
            pltpu.SemaphoreType.DMA,
            pltpu.SemaphoreType.DMA,
            pltpu.SemaphoreType.DMA,
            pltpu.SemaphoreType.DMA,
        ],
    )
    def k(hn_hbm, nrm_hbm, beta_hbm, src_hbm, dst_hbm, acc_hbm, den_hbm,
          nrm_v, src_v, dst_v, hsA, hdA, msgA, wA, hsB, hdB, msgB, wB,
          z_v, zd_v, beta_v, tab_sh, acc_sh, den_sh,
          semgA, semgB, semsA, semsB):
        cid = lax.axis_index("c")
        sid = lax.axis_index("s")
        wid = cid * NS + sid
        my_row0 = wid * rows_per_tile

        pltpu.sync_copy(nrm_hbm, nrm_v)
        pltpu.sync_copy(beta_hbm, beta_v)
        pltpu.sync_copy(src_hbm.at[pl.ds(my_row0, rows_per_tile)], src_v)
        pltpu.sync_copy(dst_hbm.at[pl.ds(my_row0, rows_per_tile)], dst_v)

        @pl.loop(0, stripe)
        def _(r):
            z_v[r, :] = jnp.zeros((L,), jnp.float32)

        @pl.loop(0, stripe, step=L)
        def _(i):
            zd_v[pl.ds(i, L)] = jnp.zeros((L,), jnp.float32)

        base_row = sid * stripe
        pltpu.sync_copy(z_v, acc_sh.at[pl.ds(base_row, stripe)])
        pltpu.sync_copy(zd_v, den_sh.at[pl.ds(base_row, stripe)])
        pltpu.sync_copy(hn_hbm.at[pl.ds(base_row, stripe)],
                        tab_sh.at[pl.ds(base_row, stripe)])
        plsc.subcore_barrier()

        lane = lax.iota(jnp.int32, L)
        b16 = beta_v[0, pl.ds(0, L)]

        def issue_gathers(t, hs, hd, semg):
            for g in range(GROWS):
                pltpu.async_copy(tab_sh.at[src_v.at[t * GROWS + g]],
                                 hs.at[pl.ds(g * 128, 128)], semg)
                pltpu.async_copy(tab_sh.at[dst_v.at[t * GROWS + g]],
                                 hd.at[pl.ds(g * 128, 128)], semg)

        def wait_gathers(hs, hd, semg):
            pltpu.make_async_copy(hn_hbm.at[pl.ds(0, C)], hs, semg).wait()
            pltpu.make_async_copy(hn_hbm.at[pl.ds(0, C)], hd, semg).wait()

        def issue_scatters(t, msg, wv, sems):
            for g in range(GROWS):
                pltpu.async_copy(msg.at[pl.ds(g * 128, 128)],
                                 acc_sh.at[dst_v.at[t * GROWS + g]],
                                 sems, add=True)
                pltpu.async_copy(wv.at[pl.ds(g * 128, 128)],
                                 den_sh.at[dst_v.at[t * GROWS + g]],
                                 sems, add=True)

        def wait_scatters(msg, wv, sems):
            pltpu.make_async_copy(hn_hbm.at[pl.ds(0, C)], msg, sems).wait()
            pltpu.make_async_copy(nrm_hbm.at[pl.ds(0, C)], wv, sems).wait()

        def compute(t, hs, hd, msg, wv):
            @pl.loop(0, C // L)
            def _(q):
                r0 = q * L
                rows = r0 + lane
                acc = jnp.zeros((L,), jnp.float32)
                a_list = []
                for f in range(hid):
                    col = jnp.full((L,), f, jnp.int32)
                    a = plsc.load_gather(hs, [rows, col])
                    b = plsc.load_gather(hd, [rows, col])
                    a_list.append(a)
                    acc = acc + a * b
                w16 = jnp.exp(acc * b16)
                lrow = t * GROWS + q // (128 // L)
                c0 = (q % (128 // L)) * L
                src16 = src_v[lrow, pl.ds(c0, L)]
                ns16 = plsc.load_gather(nrm_v, [src16])
                v16 = w16 * ns16
                wv[pl.ds(r0, L)] = w16
                for f in range(hid):
                    col = jnp.full((L,), f, jnp.int32)
                    plsc.store_scatter(msg, [rows, col], v16 * a_list[f])

        issue_gathers(0, hsA, hdA, semgA)
        issue_gathers(1, hsB, hdB, semgB)

        @pl.loop(0, pairs)
        def _(tt):
            t0 = 2 * tt
            t1 = t0 + 1

            wait_gathers(hsA, hdA, semgA)

            @pl.when(tt > 0)
            def _():
                wait_scatters(msgA, wA, semsA)

            compute(t0, hsA, hdA, msgA, wA)
            issue_scatters(t0, msgA, wA, semsA)

            @pl.when(tt < pairs - 1)
            def _():
                issue_gathers(t0 + 2, hsA, hdA, semgA)

            wait_gathers(hsB, hdB, semgB)

            @pl.when(tt > 0)
            def _():
                wait_scatters(msgB, wB, semsB)

            compute(t1, hsB, hdB, msgB, wB)
            issue_scatters(t1, msgB, wB, semsB)

            @pl.when(tt < pairs - 1)
            def _():
                issue_gathers(t1 + 2, hsB, hdB, semgB)

        wait_scatters(msgA, wA, semsA)
        wait_scatters(msgB, wB, semsB)

        plsc.subcore_barrier()
        pltpu.sync_copy(acc_sh.at[pl.ds(base_row, stripe)],
                        acc_hbm.at[cid, pl.ds(base_row, stripe)])
        pltpu.sync_copy(den_sh.at[pl.ds(base_row, stripe)],
                        den_hbm.at[cid, pl.ds(base_row, stripe)])

    return k(hn, nrm, beta_row, srcm, dstm)


def kernel(x, edge_index, W1, b1, beta1, beta2, beta3, W2, b2):
    n, nfeat = x.shape
    e = edge_index.shape[1]
    hid = W1.shape[1]

    npad = ((n + 16) + NW * L - 1) // (NW * L) * (NW * L)
    etot = e + n
    epad = (etot + 2 * NW * C - 1) // (2 * NW * C) * (2 * NW * C)
    erows = epad // 128

    loop_idx = jnp.arange(n, dtype=jnp.int32)
    pad_idx = jnp.arange(epad - etot, dtype=jnp.int32)
    src = jnp.concatenate([edge_index[0].astype(jnp.int32), loop_idx,
                           pad_idx % n])
    dst = jnp.concatenate([edge_index[1].astype(jnp.int32), loop_idx,
                           n + pad_idx % (npad - n)])
    srcm = src.reshape(erows, 128)
    dstm = dst.reshape(erows, 128)

    x_pad = jnp.pad(x, ((0, npad - n), (0, 0)))
    b1_2d = b1.reshape(1, hid)
    beta1_2d = jnp.reshape(beta1, (1, 1)).astype(jnp.float32)
    beta2_2d = jnp.reshape(beta2, (1, 1)).astype(jnp.float32)
    beta3_2d = jnp.reshape(beta3, (1, 1)).astype(jnp.float32)

    hn, nrm2, beta_row = _input_stage(x_pad, W1, b1_2d, beta1_2d, n, npad)
    acc, den = _agnn_sparse_pass(hn, nrm2.reshape(npad), beta_row, srcm, dstm)

    hn, nrm2, beta_row = _combine_stage(acc, den, beta2_2d)
    acc, den = _agnn_sparse_pass(hn, nrm2.reshape(npad), beta_row, srcm, dstm)

    hn, nrm2, beta_row = _combine_stage(acc, den, beta3_2d)
    acc, den = _agnn_sparse_pass(hn, nrm2.reshape(npad), beta_row, srcm, dstm)

    out = _output_stage(acc, den, W2, b2.reshape(1, -1))
    return out[:n]


# speedup vs baseline: 46.9579x; 1.0233x over previous
import dataclasses
import functools

import jax
import jax.numpy as jnp
from jax import lax
from jax.experimental import pallas as pl
from jax.experimental.pallas import tpu as pltpu
from jax.experimental.pallas import tpu_sc as plsc

NC = 2
NS = 16
L = 16
NW = NC * NS

GROWS = 6
C = GROWS * 128


def _input_stage(x_pad, W1, b1_2d, beta_2d, n_real, npad):
    def body(x_ref, w_ref, b_ref, beta_ref, hn_ref, nrm_ref, beta_row_ref):
        h = jnp.dot(x_ref[...], w_ref[...], preferred_element_type=jnp.float32)
        h = jnp.maximum(h + b_ref[...], 0.0)
        rows = lax.broadcasted_iota(jnp.int32, h.shape, 0)
        h = jnp.where(rows < n_real, h, 0.0)
        nrm = jnp.sqrt(jnp.sum(h * h, axis=1, keepdims=True))
        hn = h / jnp.maximum(nrm, 1e-12)
        hn_ref[...] = hn
        nrm_ref[...] = nrm
        beta_row_ref[...] = jnp.broadcast_to(beta_ref[0, 0], (1, 128))

    hid = W1.shape[1]
    return pl.pallas_call(
        body,
        out_shape=[
            jax.ShapeDtypeStruct((npad, hid), jnp.float32),
            jax.ShapeDtypeStruct((npad, 1), jnp.float32),
            jax.ShapeDtypeStruct((1, 128), jnp.float32),
        ],
    )(x_pad, W1, b1_2d, beta_2d)


def _combine_stage(acc, den, beta_2d):
    def body(acc_ref, den_ref, beta_ref, hn_ref, nrm_ref, beta_row_ref):
        A = acc_ref[0] + acc_ref[1]
        d = den_ref[0] + den_ref[1]
        nA = jnp.sqrt(jnp.sum(A * A, axis=1, keepdims=True))
        hn = A / jnp.maximum(nA, 1e-12)
        hn_ref[...] = hn
        nrm_ref[...] = nA / jnp.maximum(d, 1e-30)
        beta_row_ref[...] = jnp.broadcast_to(beta_ref[0, 0], (1, 128))

    npad, hid = acc.shape[1], acc.shape[2]
    return pl.pallas_call(
        body,
        out_shape=[
            jax.ShapeDtypeStruct((npad, hid), jnp.float32),
            jax.ShapeDtypeStruct((npad, 1), jnp.float32),
            jax.ShapeDtypeStruct((1, 128), jnp.float32),
        ],
    )(acc, den.reshape(NC, npad, 1), beta_2d)


def _output_stage(acc, den, W2, b2_2d):
    def body(acc_ref, den_ref, w_ref, b_ref, out_ref):
        A = acc_ref[0] + acc_ref[1]
        d = den_ref[0] + den_ref[1]
        h = A / jnp.maximum(d, 1e-30)
        logits = jnp.dot(h, w_ref[...], preferred_element_type=jnp.float32)
        logits = logits + b_ref[...]
        m = jnp.max(logits, axis=1, keepdims=True)
        e = jnp.exp(logits - m)
        out_ref[...] = e / jnp.sum(e, axis=1, keepdims=True)

    npad = acc.shape[1]
    ncls = W2.shape[1]
    return pl.pallas_call(
        body,
        out_shape=jax.ShapeDtypeStruct((npad, ncls), jnp.float32),
    )(acc, den.reshape(NC, npad, 1), W2, b2_2d)


def _agnn_sparse_pass(hn, nrm, beta_row, srcm, dstm):
    npad, hid = hn.shape
    erows = srcm.shape[0]
    rows_per_tile = erows // NW
    chunks = rows_per_tile // GROWS
    pairs = chunks // 2
    stripe = npad // NS

    mesh = plsc.VectorSubcoreMesh(core_axis_name="c", subcore_axis_name="s")

    cp = pltpu.CompilerParams()
    for fld, val in (("needs_layout_passes", False),
                     ("use_tc_tiling_on_sc", False)):
        if fld in pltpu.CompilerParams.__dataclass_fields__:
            cp = dataclasses.replace(cp, **{fld: val})

    @functools.partial(
        pl.kernel,
        compiler_params=cp,
        out_type=[
            jax.ShapeDtypeStruct((NC, npad, hid), jnp.float32),
            jax.ShapeDtypeStruct((NC, npad), jnp.float32),
        ],
        mesh=mesh,
        scratch_types=[
            pltpu.VMEM((npad,), jnp.float32),
            pltpu.VMEM((rows_per_tile, 128), jnp.int32),
            pltpu.VMEM((rows_per_tile, 128), jnp.int32),
            pltpu.VMEM((C, hid), jnp.float32),
            pltpu.VMEM((C, hid), jnp.float32),
            pltpu.VMEM((C, hid), jnp.float32),
            pltpu.VMEM((C,), jnp.float32),
            pltpu.VMEM((C, hid), jnp.float32),
            pltpu.VMEM((C, hid), jnp.float32),
            pltpu.VMEM((C, hid), jnp.float32),
            pltpu.VMEM((C,), jnp.float32),
            pltpu.VMEM((1, 128), jnp.float32),
            pltpu.VMEM_SHARED((npad, hid), jnp.float32),
            pltpu.VMEM_SHARED((npad, hid), jnp.float32),
            pltpu.VMEM_SHARED((npad,), jnp.float32),
            pltpu.SemaphoreType.DMA,
            pltpu.SemaphoreType.DMA,
            pltpu.SemaphoreType.DMA,
            pltpu.SemaphoreType.DMA,
        ],
    )
    def k(hn_hbm, nrm_hbm, beta_hbm, src_hbm, dst_hbm, acc_hbm, den_hbm,
          nrm_v, src_v, dst_v, hsA, hdA, msgA, wA, hsB, hdB, msgB, wB,
          beta_v, tab_sh, acc_sh, den_sh,
          semgA, semgB, semsA, semsB):
        cid = lax.axis_index("c")
        sid = lax.axis_index("s")
        wid = cid * NS + sid
        my_row0 = wid * rows_per_tile

        pltpu.sync_copy(nrm_hbm, nrm_v)
        pltpu.sync_copy(beta_hbm, beta_v)
        pltpu.sync_copy(src_hbm.at[pl.ds(my_row0, rows_per_tile)], src_v)
        pltpu.sync_copy(dst_hbm.at[pl.ds(my_row0, rows_per_tile)], dst_v)

        @pl.loop(0, stripe)
        def _(r):
            msgA[r, :] = jnp.zeros((L,), jnp.float32)

        @pl.loop(0, stripe, step=L)
        def _(i):
            wA[pl.ds(i, L)] = jnp.zeros((L,), jnp.float32)

        base_row = sid * stripe
        pltpu.sync_copy(msgA.at[pl.ds(0, stripe)],
                        acc_sh.at[pl.ds(base_row, stripe)])
        pltpu.sync_copy(wA.at[pl.ds(0, stripe)],
                        den_sh.at[pl.ds(base_row, stripe)])
        pltpu.sync_copy(hn_hbm.at[pl.ds(base_row, stripe)],
                        tab_sh.at[pl.ds(base_row, stripe)])
        plsc.subcore_barrier()

        lane = lax.iota(jnp.int32, L)
        b16 = beta_v[0, pl.ds(0, L)]

        def issue_gathers(t, hs, hd, semg):
            for g in range(GROWS):
                pltpu.async_copy(tab_sh.at[src_v.at[t * GROWS + g]],
                                 hs.at[pl.ds(g * 128, 128)], semg)
                pltpu.async_copy(tab_sh.at[dst_v.at[t * GROWS + g]],
                                 hd.at[pl.ds(g * 128, 128)], semg)

        def wait_gathers(hs, hd, semg):
            pltpu.make_async_copy(hn_hbm.at[pl.ds(0, C)], hs, semg).wait()
            pltpu.make_async_copy(hn_hbm.at[pl.ds(0, C)], hd, semg).wait()

        def issue_scatters(t, msg, wv, sems):
            for g in range(GROWS):
                pltpu.async_copy(msg.at[pl.ds(g * 128, 128)],
                                 acc_sh.at[dst_v.at[t * GROWS + g]],
                                 sems, add=True)
                pltpu.async_copy(wv.at[pl.ds(g * 128, 128)],
                                 den_sh.at[dst_v.at[t * GROWS + g]],
                                 sems, add=True)

        def wait_scatters(msg, wv, sems):
            pltpu.make_async_copy(hn_hbm.at[pl.ds(0, C)], msg, sems).wait()
            pltpu.make_async_copy(nrm_hbm.at[pl.ds(0, C)], wv, sems).wait()

        def compute(t, hs, hd, msg, wv):
            @pl.loop(0, C // L)
            def _(q):
                r0 = q * L
                rows = r0 + lane
                acc = jnp.zeros((L,), jnp.float32)
                a_list = []
                for f in range(hid):
                    col = jnp.full((L,), f, jnp.int32)
                    a = plsc.load_gather(hs, [rows, col])
                    b = plsc.load_gather(hd, [rows, col])
                    a_list.append(a)
                    acc = acc + a * b
                w16 = jnp.exp(acc * b16)
                lrow = t * GROWS + q // (128 // L)
                c0 = (q % (128 // L)) * L
                src16 = src_v[lrow, pl.ds(c0, L)]
                ns16 = plsc.load_gather(nrm_v, [src16])
                v16 = w16 * ns16
                wv[pl.ds(r0, L)] = w16
                for f in range(hid):
                    col = jnp.full((L,), f, jnp.int32)
                    plsc.store_scatter(msg, [rows, col], v16 * a_list[f])

        issue_gathers(0, hsA, hdA, semgA)
        issue_gathers(1, hsB, hdB, semgB)

        @pl.loop(0, pairs)
        def _(tt):
            t0 = 2 * tt
            t1 = t0 + 1

            wait_gathers(hsA, hdA, semgA)

            @pl.when(tt > 0)
            def _():
                wait_scatters(msgA, wA, semsA)

            compute(t0, hsA, hdA, msgA, wA)
            issue_scatters(t0, msgA, wA, semsA)

            @pl.when(tt < pairs - 1)
            def _():
                issue_gathers(t0 + 2, hsA, hdA, semgA)

            wait_gathers(hsB, hdB, semgB)

            @pl.when(tt > 0)
            def _():
                wait_scatters(msgB, wB, semsB)

            compute(t1, hsB, hdB, msgB, wB)
            issue_scatters(t1, msgB, wB, semsB)

            @pl.when(tt < pairs - 1)
            def _():
                issue_gathers(t1 + 2, hsB, hdB, semgB)

        wait_scatters(msgA, wA, semsA)
        wait_scatters(msgB, wB, semsB)

        plsc.subcore_barrier()
        pltpu.sync_copy(acc_sh.at[pl.ds(base_row, stripe)],
                        acc_hbm.at[cid, pl.ds(base_row, stripe)])
        pltpu.sync_copy(den_sh.at[pl.ds(base_row, stripe)],
                        den_hbm.at[cid, pl.ds(base_row, stripe)])

    return k(hn, nrm, beta_row, srcm, dstm)


def kernel(x, edge_index, W1, b1, beta1, beta2, beta3, W2, b2):
    n, nfeat = x.shape
    e = edge_index.shape[1]
    hid = W1.shape[1]

    npad = ((n + 16) + NW * L - 1) // (NW * L) * (NW * L)
    etot = e + n
    epad = (etot + 2 * NW * C - 1) // (2 * NW * C) * (2 * NW * C)
    erows = epad // 128

    loop_idx = jnp.arange(n, dtype=jnp.int32)
    pad_idx = jnp.arange(epad - etot, dtype=jnp.int32)
    src = jnp.concatenate([edge_index[0].astype(jnp.int32), loop_idx,
                           pad_idx % n])
    dst = jnp.concatenate([edge_index[1].astype(jnp.int32), loop_idx,
                           n + pad_idx % (npad - n)])
    srcm = src.reshape(erows, 128)
    dstm = dst.reshape(erows, 128)

    x_pad = jnp.pad(x, ((0, npad - n), (0, 0)))
    b1_2d = b1.reshape(1, hid)
    beta1_2d = jnp.reshape(beta1, (1, 1)).astype(jnp.float32)
    beta2_2d = jnp.reshape(beta2, (1, 1)).astype(jnp.float32)
    beta3_2d = jnp.reshape(beta3, (1, 1)).astype(jnp.float32)

    hn, nrm2, beta_row = _input_stage(x_pad, W1, b1_2d, beta1_2d, n, npad)
    acc, den = _agnn_sparse_pass(hn, nrm2.reshape(npad), beta_row, srcm, dstm)

    hn, nrm2, beta_row = _combine_stage(acc, den, beta2_2d)
    acc, den = _agnn_sparse_pass(hn, nrm2.reshape(npad), beta_row, srcm, dstm)

    hn, nrm2, beta_row = _combine_stage(acc, den, beta3_2d)
    acc, den = _agnn_sparse_pass(hn, nrm2.reshape(npad), beta_row, srcm, dstm)

    out = _output_stage(acc, den, W2, b2.reshape(1, -1))
    return out[:n]


# speedup vs baseline: 52.9712x; 1.1281x over previous
import dataclasses
import functools

import jax
import jax.numpy as jnp
from jax import lax
from jax.experimental import pallas as pl
from jax.experimental.pallas import tpu as pltpu
from jax.experimental.pallas import tpu_sc as plsc

NC = 2
NS = 16
L = 16
NW = NC * NS

GROWS = 6
C = GROWS * 128


def _input_stage(x_pad, W1, b1_2d, beta_2d, n_real, npad):
    def body(x_ref, w_ref, b_ref, beta_ref, hn_ref, nrm_ref, beta_rows_ref):
        h = jnp.dot(x_ref[...], w_ref[...], preferred_element_type=jnp.float32)
        h = jnp.maximum(h + b_ref[...], 0.0)
        rows = lax.broadcasted_iota(jnp.int32, h.shape, 0)
        h = jnp.where(rows < n_real, h, 0.0)
        nrm = jnp.sqrt(jnp.sum(h * h, axis=1, keepdims=True))
        hn = h / jnp.maximum(nrm, 1e-12)
        hn_ref[...] = hn
        nrm_ref[...] = nrm
        beta_rows_ref[...] = jnp.broadcast_to(beta_ref[...], (3, 128))

    hid = W1.shape[1]
    return pl.pallas_call(
        body,
        out_shape=[
            jax.ShapeDtypeStruct((npad, hid), jnp.float32),
            jax.ShapeDtypeStruct((npad, 1), jnp.float32),
            jax.ShapeDtypeStruct((3, 128), jnp.float32),
        ],
    )(x_pad, W1, b1_2d, beta_2d)


def _output_stage(acc, den, W2, b2_2d):
    def body(acc_ref, den_ref, w_ref, b_ref, out_ref):
        A = acc_ref[0] + acc_ref[1]
        d = den_ref[0] + den_ref[1]
        h = A / jnp.maximum(d, 1e-30)
        logits = jnp.dot(h, w_ref[...], preferred_element_type=jnp.float32)
        logits = logits + b_ref[...]
        m = jnp.max(logits, axis=1, keepdims=True)
        e = jnp.exp(logits - m)
        out_ref[...] = e / jnp.sum(e, axis=1, keepdims=True)

    npad = acc.shape[1]
    ncls = W2.shape[1]
    return pl.pallas_call(
        body,
        out_shape=jax.ShapeDtypeStruct((npad, ncls), jnp.float32),
    )(acc, den.reshape(NC, npad, 1), W2, b2_2d)


def _rsqrt16(x):
    i = plsc.bitcast(x, jnp.int32)
    i = jnp.full((L,), 0x5F3759DF, jnp.int32) - lax.shift_right_logical(
        i, jnp.full((L,), 1, jnp.int32))
    y = plsc.bitcast(i, jnp.float32)
    for _ in range(3):
        y = y * (1.5 - 0.5 * x * y * y)
    return y


def _agnn_sparse_pass(a1, a2, beta_row, srcm, dstm, first):
    npad, hid = a1.shape[-2], a1.shape[-1]
    erows = srcm.shape[0]
    rows_per_tile = erows // NW
    chunks = rows_per_tile // GROWS
    pairs = chunks // 2
    stripe = npad // NS

    mesh = plsc.VectorSubcoreMesh(core_axis_name="c", subcore_axis_name="s")

    cp = pltpu.CompilerParams()
    for fld, val in (("needs_layout_passes", False),
                     ("use_tc_tiling_on_sc", False)):
        if fld in pltpu.CompilerParams.__dataclass_fields__:
            cp = dataclasses.replace(cp, **{fld: val})

    @functools.partial(
        pl.kernel,
        compiler_params=cp,
        out_type=[
            jax.ShapeDtypeStruct((NC, npad, hid), jnp.float32),
            jax.ShapeDtypeStruct((NC, npad), jnp.float32),
        ],
        mesh=mesh,
        scratch_types=[
            pltpu.VMEM((npad,), jnp.float32),
            pltpu.VMEM((rows_per_tile, 128), jnp.int32),
            pltpu.VMEM((rows_per_tile, 128), jnp.int32),
            pltpu.VMEM((C, hid), jnp.float32),
            pltpu.VMEM((C, hid), jnp.float32),
            pltpu.VMEM((C, hid), jnp.float32),
            pltpu.VMEM((C,), jnp.float32),
            pltpu.VMEM((C, hid), jnp.float32),
            pltpu.VMEM((C, hid), jnp.float32),
            pltpu.VMEM((C, hid), jnp.float32),
            pltpu.VMEM((C,), jnp.float32),
            pltpu.VMEM((1, 128), jnp.float32),
            pltpu.VMEM((npad // NS,), jnp.float32),
            pltpu.VMEM((npad // NS,), jnp.float32),
            pltpu.VMEM_SHARED((npad, hid), jnp.float32),
            pltpu.VMEM_SHARED((npad,), jnp.float32),
            pltpu.VMEM_SHARED((npad, hid), jnp.float32),
            pltpu.VMEM_SHARED((npad,), jnp.float32),
            pltpu.SemaphoreType.DMA,
            pltpu.SemaphoreType.DMA,
            pltpu.SemaphoreType.DMA,
            pltpu.SemaphoreType.DMA,
        ],
    )
    def k(a1_hbm, a2_hbm, beta_hbm, src_hbm, dst_hbm, acc_hbm, den_hbm,
          nrm_v, src_v, dst_v, hsA, hdA, msgA, wA, hsB, hdB, msgB, wB,
          beta_v, d0_v, d1_v, tab_sh, nrm_sh, acc_sh, den_sh,
          semgA, semgB, semsA, semsB):
        cid = lax.axis_index("c")
        sid = lax.axis_index("s")
        wid = cid * NS + sid
        my_row0 = wid * rows_per_tile
        base_row = sid * stripe
        lane = lax.iota(jnp.int32, L)

        pltpu.sync_copy(beta_hbm, beta_v)
        pltpu.sync_copy(src_hbm.at[pl.ds(my_row0, rows_per_tile)], src_v)
        pltpu.sync_copy(dst_hbm.at[pl.ds(my_row0, rows_per_tile)], dst_v)

        if first:
            pltpu.sync_copy(a2_hbm, nrm_v)
            pltpu.sync_copy(a1_hbm.at[pl.ds(base_row, stripe)],
                            tab_sh.at[pl.ds(base_row, stripe)])
        else:
            pltpu.sync_copy(a2_hbm.at[0, pl.ds(base_row, stripe)], d0_v)
            pltpu.sync_copy(a2_hbm.at[1, pl.ds(base_row, stripe)], d1_v)
            pltpu.sync_copy(a1_hbm.at[0, pl.ds(base_row, stripe)],
                            hsA.at[pl.ds(0, stripe)])
            pltpu.sync_copy(a1_hbm.at[1, pl.ds(base_row, stripe)],
                            hsB.at[pl.ds(0, stripe)])

            @pl.loop(0, stripe // L)
            def _(q):
                rows = q * L + lane
                nrm2 = jnp.zeros((L,), jnp.float32)
                a_list = []
                for f in range(hid):
                    col = jnp.full((L,), f, jnp.int32)
                    a = (plsc.load_gather(hsA, [rows, col]) +
                         plsc.load_gather(hsB, [rows, col]))
                    a_list.append(a)
                    nrm2 = nrm2 + a * a
                y = _rsqrt16(jnp.maximum(nrm2, 1e-24))
                d16 = d0_v[pl.ds(q * L, L)] + d1_v[pl.ds(q * L, L)]
                wA[pl.ds(q * L, L)] = (nrm2 * y) / jnp.maximum(d16, 1e-30)
                for f in range(hid):
                    col = jnp.full((L,), f, jnp.int32)
                    plsc.store_scatter(msgA, [rows, col], a_list[f] * y)

            pltpu.sync_copy(msgA.at[pl.ds(0, stripe)],
                            tab_sh.at[pl.ds(base_row, stripe)])
            pltpu.sync_copy(wA.at[pl.ds(0, stripe)],
                            nrm_sh.at[pl.ds(base_row, stripe)])

        @pl.loop(0, stripe)
        def _(r):
            msgA[r, :] = jnp.zeros((L,), jnp.float32)

        @pl.loop(0, stripe, step=L)
        def _(i):
            wA[pl.ds(i, L)] = jnp.zeros((L,), jnp.float32)

        pltpu.sync_copy(msgA.at[pl.ds(0, stripe)],
                        acc_sh.at[pl.ds(base_row, stripe)])
        pltpu.sync_copy(wA.at[pl.ds(0, stripe)],
                        den_sh.at[pl.ds(base_row, stripe)])
        plsc.subcore_barrier()
        if not first:
            pltpu.sync_copy(nrm_sh, nrm_v)

        b16 = beta_v[0, pl.ds(0, L)]

        def issue_gathers(t, hs, hd, semg):
            for g in range(GROWS):
                pltpu.async_copy(tab_sh.at[src_v.at[t * GROWS + g]],
                                 hs.at[pl.ds(g * 128, 128)], semg)
                pltpu.async_copy(tab_sh.at[dst_v.at[t * GROWS + g]],
                                 hd.at[pl.ds(g * 128, 128)], semg)

        def wait_gathers(hs, hd, semg):
            pltpu.make_async_copy(acc_hbm.at[0, pl.ds(0, C)], hs, semg).wait()
            pltpu.make_async_copy(acc_hbm.at[0, pl.ds(0, C)], hd, semg).wait()

        def issue_scatters(t, msg, wv, sems):
            for g in range(GROWS):
                pltpu.async_copy(msg.at[pl.ds(g * 128, 128)],
                                 acc_sh.at[dst_v.at[t * GROWS + g]],
                                 sems, add=True)
                pltpu.async_copy(wv.at[pl.ds(g * 128, 128)],
                                 den_sh.at[dst_v.at[t * GROWS + g]],
                                 sems, add=True)

        def wait_scatters(msg, wv, sems):
            pltpu.make_async_copy(acc_hbm.at[0, pl.ds(0, C)], msg, sems).wait()
            pltpu.make_async_copy(den_hbm.at[0, pl.ds(0, C)], wv, sems).wait()

        def compute(t, hs, hd, msg, wv):
            @pl.loop(0, C // L)
            def _(q):
                r0 = q * L
                rows = r0 + lane
                acc = jnp.zeros((L,), jnp.float32)
                a_list = []
                for f in range(hid):
                    col = jnp.full((L,), f, jnp.int32)
                    a = plsc.load_gather(hs, [rows, col])
                    b = plsc.load_gather(hd, [rows, col])
                    a_list.append(a)
                    acc = acc + a * b
                w16 = jnp.exp(acc * b16)
                lrow = t * GROWS + q // (128 // L)
                c0 = (q % (128 // L)) * L
                src16 = src_v[lrow, pl.ds(c0, L)]
                ns16 = plsc.load_gather(nrm_v, [src16])
                v16 = w16 * ns16
                wv[pl.ds(r0, L)] = w16
                for f in range(hid):
                    col = jnp.full((L,), f, jnp.int32)
                    plsc.store_scatter(msg, [rows, col], v16 * a_list[f])

        issue_gathers(0, hsA, hdA, semgA)
        issue_gathers(1, hsB, hdB, semgB)

        @pl.loop(0, pairs)
        def _(tt):
            t0 = 2 * tt
            t1 = t0 + 1

            wait_gathers(hsA, hdA, semgA)

            @pl.when(tt > 0)
            def _():
                wait_scatters(msgA, wA, semsA)

            compute(t0, hsA, hdA, msgA, wA)
            issue_scatters(t0, msgA, wA, semsA)

            @pl.when(tt < pairs - 1)
            def _():
                issue_gathers(t0 + 2, hsA, hdA, semgA)

            wait_gathers(hsB, hdB, semgB)

            @pl.when(tt > 0)
            def _():
                wait_scatters(msgB, wB, semsB)

            compute(t1, hsB, hdB, msgB, wB)
            issue_scatters(t1, msgB, wB, semsB)

            @pl.when(tt < pairs - 1)
            def _():
                issue_gathers(t1 + 2, hsB, hdB, semgB)

        wait_scatters(msgA, wA, semsA)
        wait_scatters(msgB, wB, semsB)

        plsc.subcore_barrier()
        pltpu.sync_copy(acc_sh.at[pl.ds(base_row, stripe)],
                        acc_hbm.at[cid, pl.ds(base_row, stripe)])
        pltpu.sync_copy(den_sh.at[pl.ds(base_row, stripe)],
                        den_hbm.at[cid, pl.ds(base_row, stripe)])

    return k(a1, a2, beta_row, srcm, dstm)


def kernel(x, edge_index, W1, b1, beta1, beta2, beta3, W2, b2):
    n, nfeat = x.shape
    e = edge_index.shape[1]
    hid = W1.shape[1]

    npad = ((n + 16) + NW * L - 1) // (NW * L) * (NW * L)
    etot = e + n
    epad = (etot + 2 * NW * C - 1) // (2 * NW * C) * (2 * NW * C)
    erows = epad // 128

    loop_idx = jnp.arange(n, dtype=jnp.int32)
    pad_idx = jnp.arange(epad - etot, dtype=jnp.int32)
    src = jnp.concatenate([edge_index[0].astype(jnp.int32), loop_idx,
                           pad_idx % n])
    dst = jnp.concatenate([edge_index[1].astype(jnp.int32), loop_idx,
                           n + pad_idx % (npad - n)])
    srcm = src.reshape(erows, 128)
    dstm = dst.reshape(erows, 128)

    x_pad = jnp.pad(x, ((0, npad - n), (0, 0)))
    b1_2d = b1.reshape(1, hid)
    betas31 = jnp.stack([beta1, beta2, beta3]).astype(jnp.float32).reshape(3, 1)

    hn, nrm2, beta_rows = _input_stage(x_pad, W1, b1_2d, betas31, n, npad)
    acc, den = _agnn_sparse_pass(hn, nrm2.reshape(npad), beta_rows[0:1],
                                 srcm, dstm, True)
    acc, den = _agnn_sparse_pass(acc, den, beta_rows[1:2], srcm, dstm, False)
    acc, den = _agnn_sparse_pass(acc, den, beta_rows[2:3], srcm, dstm, False)

    out = _output_stage(acc, den, W2, b2.reshape(1, -1))
    return out[:n]


# speedup vs baseline: 86.3426x; 1.6300x over previous
import dataclasses
import functools

import jax
import jax.numpy as jnp
from jax import lax
from jax.experimental import pallas as pl
from jax.experimental.pallas import tpu as pltpu
from jax.experimental.pallas import tpu_sc as plsc

NC = 2
NS = 16
L = 16
NW = NC * NS

GROWS = 6
C = GROWS * 128


def _input_stage(x_pad, W1, b1_2d, beta_2d, n_real, npad):
    def body(x_ref, w_ref, b_ref, beta_ref, hn_ref, nrm_ref, beta_rows_ref):
        h = jnp.dot(x_ref[...], w_ref[...], preferred_element_type=jnp.float32)
        h = jnp.maximum(h + b_ref[...], 0.0)
        rows = lax.broadcasted_iota(jnp.int32, h.shape, 0)
        h = jnp.where(rows < n_real, h, 0.0)
        nrm = jnp.sqrt(jnp.sum(h * h, axis=1, keepdims=True))
        hn = h / jnp.maximum(nrm, 1e-12)
        hn_ref[...] = hn
        nrm_ref[...] = nrm
        beta_rows_ref[...] = jnp.broadcast_to(beta_ref[...], (3, 128))

    hid = W1.shape[1]
    return pl.pallas_call(
        body,
        out_shape=[
            jax.ShapeDtypeStruct((npad, hid), jnp.float32),
            jax.ShapeDtypeStruct((npad, 1), jnp.float32),
            jax.ShapeDtypeStruct((3, 128), jnp.float32),
        ],
    )(x_pad, W1, b1_2d, beta_2d)


def _output_stage(acc, den, W2, b2_2d):
    def body(acc_ref, den_ref, w_ref, b_ref, out_ref):
        A = acc_ref[0] + acc_ref[1]
        d = den_ref[0] + den_ref[1]
        h = A / jnp.maximum(d, 1e-30)
        logits = jnp.dot(h, w_ref[...], preferred_element_type=jnp.float32)
        logits = logits + b_ref[...]
        m = jnp.max(logits, axis=1, keepdims=True)
        e = jnp.exp(logits - m)
        out_ref[...] = e / jnp.sum(e, axis=1, keepdims=True)

    npad = acc.shape[1]
    ncls = W2.shape[1]
    return pl.pallas_call(
        body,
        out_shape=jax.ShapeDtypeStruct((npad, ncls), jnp.float32),
    )(acc, den.reshape(NC, npad, 1), W2, b2_2d)


def _rsqrt16(x):
    i = plsc.bitcast(x, jnp.int32)
    i = jnp.full((L,), 0x5F3759DF, jnp.int32) - lax.shift_right_logical(
        i, jnp.full((L,), 1, jnp.int32))
    y = plsc.bitcast(i, jnp.float32)
    for _ in range(3):
        y = y * (1.5 - 0.5 * x * y * y)
    return y


def _agnn_sparse_pass(a1, a2, beta_row, srcm, dstm, first):
    npad, hid = a1.shape[-2], a1.shape[-1]
    erows = srcm.shape[0]
    rows_per_tile = erows // NW
    chunks = rows_per_tile // GROWS
    pairs = chunks // 2
    stripe = npad // NS

    mesh = plsc.VectorSubcoreMesh(core_axis_name="c", subcore_axis_name="s")

    cp = pltpu.CompilerParams()
    for fld, val in (("needs_layout_passes", False),
                     ("use_tc_tiling_on_sc", False)):
        if fld in pltpu.CompilerParams.__dataclass_fields__:
            cp = dataclasses.replace(cp, **{fld: val})

    @functools.partial(
        pl.kernel,
        compiler_params=cp,
        out_type=[
            jax.ShapeDtypeStruct((NC, npad, hid), jnp.float32),
            jax.ShapeDtypeStruct((NC, npad), jnp.float32),
        ],
        mesh=mesh,
        scratch_types=[
            pltpu.VMEM((npad,), jnp.float32),
            pltpu.VMEM((rows_per_tile, 128), jnp.int32),
            pltpu.VMEM((rows_per_tile, 128), jnp.int32),
            pltpu.VMEM((C, hid), jnp.float32),
            pltpu.VMEM((C, hid), jnp.float32),
            pltpu.VMEM((C, hid), jnp.float32),
            pltpu.VMEM((C,), jnp.float32),
            pltpu.VMEM((C, hid), jnp.float32),
            pltpu.VMEM((C, hid), jnp.float32),
            pltpu.VMEM((C, hid), jnp.float32),
            pltpu.VMEM((C,), jnp.float32),
            pltpu.VMEM((1, 128), jnp.float32),
            pltpu.VMEM((npad // NS,), jnp.float32),
            pltpu.VMEM((npad // NS,), jnp.float32),
            pltpu.VMEM_SHARED((npad, hid), jnp.float32),
            pltpu.VMEM_SHARED((npad,), jnp.float32),
            pltpu.VMEM_SHARED((npad, hid), jnp.float32),
            pltpu.VMEM_SHARED((npad,), jnp.float32),
            pltpu.SemaphoreType.DMA,
            pltpu.SemaphoreType.DMA,
            pltpu.SemaphoreType.DMA,
            pltpu.SemaphoreType.DMA,
        ],
    )
    def k(a1_hbm, a2_hbm, beta_hbm, src_hbm, dst_hbm, acc_hbm, den_hbm,
          nrm_v, src_v, dst_v, hsA, hdA, msgA, wA, hsB, hdB, msgB, wB,
          beta_v, d0_v, d1_v, tab_sh, nrm_sh, acc_sh, den_sh,
          semgA, semgB, semsA, semsB):
        cid = lax.axis_index("c")
        sid = lax.axis_index("s")
        wid = cid * NS + sid
        my_row0 = wid * rows_per_tile
        base_row = sid * stripe
        lane = lax.iota(jnp.int32, L)

        pltpu.sync_copy(beta_hbm, beta_v)
        pltpu.sync_copy(src_hbm.at[pl.ds(my_row0, rows_per_tile)], src_v)
        pltpu.sync_copy(dst_hbm.at[pl.ds(my_row0, rows_per_tile)], dst_v)

        if first:
            pltpu.sync_copy(a2_hbm, nrm_v)
            pltpu.sync_copy(a1_hbm.at[pl.ds(base_row, stripe)],
                            tab_sh.at[pl.ds(base_row, stripe)])
        else:
            pltpu.sync_copy(a2_hbm.at[0, pl.ds(base_row, stripe)], d0_v)
            pltpu.sync_copy(a2_hbm.at[1, pl.ds(base_row, stripe)], d1_v)
            pltpu.sync_copy(a1_hbm.at[0, pl.ds(base_row, stripe)],
                            hsA.at[pl.ds(0, stripe)])
            pltpu.sync_copy(a1_hbm.at[1, pl.ds(base_row, stripe)],
                            hsB.at[pl.ds(0, stripe)])

            @pl.loop(0, stripe // L)
            def _(q):
                rows = q * L + lane
                nrm2 = jnp.zeros((L,), jnp.float32)
                a_list = []
                col_list = []
                for s in range(hid):
                    col = jnp.bitwise_and(lane + s, hid - 1)
                    a = (plsc.load_gather(hsA, [rows, col]) +
                         plsc.load_gather(hsB, [rows, col]))
                    a_list.append(a)
                    col_list.append(col)
                    nrm2 = nrm2 + a * a
                y = _rsqrt16(jnp.maximum(nrm2, 1e-24))
                d16 = d0_v[pl.ds(q * L, L)] + d1_v[pl.ds(q * L, L)]
                wA[pl.ds(q * L, L)] = (nrm2 * y) / jnp.maximum(d16, 1e-30)
                for s in range(hid):
                    plsc.store_scatter(msgA, [rows, col_list[s]],
                                       a_list[s] * y)

            pltpu.sync_copy(msgA.at[pl.ds(0, stripe)],
                            tab_sh.at[pl.ds(base_row, stripe)])
            pltpu.sync_copy(wA.at[pl.ds(0, stripe)],
                            nrm_sh.at[pl.ds(base_row, stripe)])

        @pl.loop(0, stripe)
        def _(r):
            msgA[r, :] = jnp.zeros((L,), jnp.float32)

        @pl.loop(0, stripe, step=L)
        def _(i):
            wA[pl.ds(i, L)] = jnp.zeros((L,), jnp.float32)

        pltpu.sync_copy(msgA.at[pl.ds(0, stripe)],
                        acc_sh.at[pl.ds(base_row, stripe)])
        pltpu.sync_copy(wA.at[pl.ds(0, stripe)],
                        den_sh.at[pl.ds(base_row, stripe)])
        plsc.subcore_barrier()
        if not first:
            pltpu.sync_copy(nrm_sh, nrm_v)

        b16 = beta_v[0, pl.ds(0, L)]

        def issue_gathers(t, hs, hd, semg):
            for g in range(GROWS):
                pltpu.async_copy(tab_sh.at[src_v.at[t * GROWS + g]],
                                 hs.at[pl.ds(g * 128, 128)], semg)
                pltpu.async_copy(tab_sh.at[dst_v.at[t * GROWS + g]],
                                 hd.at[pl.ds(g * 128, 128)], semg)

        def wait_gathers(hs, hd, semg):
            pltpu.make_async_copy(acc_hbm.at[0, pl.ds(0, C)], hs, semg).wait()
            pltpu.make_async_copy(acc_hbm.at[0, pl.ds(0, C)], hd, semg).wait()

        def issue_scatters(t, msg, wv, sems):
            for g in range(GROWS):
                pltpu.async_copy(msg.at[pl.ds(g * 128, 128)],
                                 acc_sh.at[dst_v.at[t * GROWS + g]],
                                 sems, add=True)
                pltpu.async_copy(wv.at[pl.ds(g * 128, 128)],
                                 den_sh.at[dst_v.at[t * GROWS + g]],
                                 sems, add=True)

        def wait_scatters(msg, wv, sems):
            pltpu.make_async_copy(acc_hbm.at[0, pl.ds(0, C)], msg, sems).wait()
            pltpu.make_async_copy(den_hbm.at[0, pl.ds(0, C)], wv, sems).wait()

        def compute(t, hs, hd, msg, wv):
            @pl.loop(0, C // L)
            def _(q):
                r0 = q * L
                rows = r0 + lane
                acc = jnp.zeros((L,), jnp.float32)
                a_list = []
                col_list = []
                for s in range(hid):
                    col = jnp.bitwise_and(lane + s, hid - 1)
                    a = plsc.load_gather(hs, [rows, col])
                    b = plsc.load_gather(hd, [rows, col])
                    a_list.append(a)
                    col_list.append(col)
                    acc = acc + a * b
                w16 = jnp.exp(acc * b16)
                lrow = t * GROWS + q // (128 // L)
                c0 = (q % (128 // L)) * L
                src16 = src_v[lrow, pl.ds(c0, L)]
                ns16 = plsc.load_gather(nrm_v, [src16])
                v16 = w16 * ns16
                wv[pl.ds(r0, L)] = w16
                for s in range(hid):
                    plsc.store_scatter(msg, [rows, col_list[s]],
                                       v16 * a_list[s])

        issue_gathers(0, hsA, hdA, semgA)
        issue_gathers(1, hsB, hdB, semgB)

        @pl.loop(0, pairs)
        def _(tt):
            t0 = 2 * tt
            t1 = t0 + 1

            wait_gathers(hsA, hdA, semgA)

            @pl.when(tt > 0)
            def _():
                wait_scatters(msgA, wA, semsA)

            compute(t0, hsA, hdA, msgA, wA)
            issue_scatters(t0, msgA, wA, semsA)

            @pl.when(tt < pairs - 1)
            def _():
                issue_gathers(t0 + 2, hsA, hdA, semgA)

            wait_gathers(hsB, hdB, semgB)

            @pl.when(tt > 0)
            def _():
                wait_scatters(msgB, wB, semsB)

            compute(t1, hsB, hdB, msgB, wB)
            issue_scatters(t1, msgB, wB, semsB)

            @pl.when(tt < pairs - 1)
            def _():
                issue_gathers(t1 + 2, hsB, hdB, semgB)

        wait_scatters(msgA, wA, semsA)
        wait_scatters(msgB, wB, semsB)

        plsc.subcore_barrier()
        pltpu.sync_copy(acc_sh.at[pl.ds(base_row, stripe)],
                        acc_hbm.at[cid, pl.ds(base_row, stripe)])
        pltpu.sync_copy(den_sh.at[pl.ds(base_row, stripe)],
                        den_hbm.at[cid, pl.ds(base_row, stripe)])

    return k(a1, a2, beta_row, srcm, dstm)


def kernel(x, edge_index, W1, b1, beta1, beta2, beta3, W2, b2):
    n, nfeat = x.shape
    e = edge_index.shape[1]
    hid = W1.shape[1]

    npad = ((n + 16) + NW * L - 1) // (NW * L) * (NW * L)
    etot = e + n
    epad = (etot + 2 * NW * C - 1) // (2 * NW * C) * (2 * NW * C)
    erows = epad // 128

    loop_idx = jnp.arange(n, dtype=jnp.int32)
    pad_idx = jnp.arange(epad - etot, dtype=jnp.int32)
    src = jnp.concatenate([edge_index[0].astype(jnp.int32), loop_idx,
                           pad_idx % n])
    dst = jnp.concatenate([edge_index[1].astype(jnp.int32), loop_idx,
                           n + pad_idx % (npad - n)])
    srcm = src.reshape(erows, 128)
    dstm = dst.reshape(erows, 128)

    x_pad = jnp.pad(x, ((0, npad - n), (0, 0)))
    b1_2d = b1.reshape(1, hid)
    betas31 = jnp.stack([beta1, beta2, beta3]).astype(jnp.float32).reshape(3, 1)

    hn, nrm2, beta_rows = _input_stage(x_pad, W1, b1_2d, betas31, n, npad)
    acc, den = _agnn_sparse_pass(hn, nrm2.reshape(npad), beta_rows[0:1],
                                 srcm, dstm, True)
    acc, den = _agnn_sparse_pass(acc, den, beta_rows[1:2], srcm, dstm, False)
    acc, den = _agnn_sparse_pass(acc, den, beta_rows[2:3], srcm, dstm, False)

    out = _output_stage(acc, den, W2, b2.reshape(1, -1))
    return out[:n]


# speedup vs baseline: 87.1750x; 1.0096x over previous
import dataclasses
import functools

import jax
import jax.numpy as jnp
from jax import lax
from jax.experimental import pallas as pl
from jax.experimental.pallas import tpu as pltpu
from jax.experimental.pallas import tpu_sc as plsc

NC = 2
NS = 16
L = 16
NW = NC * NS

GROWS = 5
C = GROWS * 128


def _input_stage(x_pad, W1, b1_2d, beta_2d, n_real, npad):
    def body(x_ref, w_ref, b_ref, beta_ref, hn_ref, nrm_ref, beta_rows_ref):
        h = jnp.dot(x_ref[...], w_ref[...], preferred_element_type=jnp.float32)
        h = jnp.maximum(h + b_ref[...], 0.0)
        rows = lax.broadcasted_iota(jnp.int32, h.shape, 0)
        h = jnp.where(rows < n_real, h, 0.0)
        nrm = jnp.sqrt(jnp.sum(h * h, axis=1, keepdims=True))
        hn = h / jnp.maximum(nrm, 1e-12)
        hn_ref[...] = hn
        nrm_ref[...] = nrm
        beta_rows_ref[...] = jnp.broadcast_to(beta_ref[...], (3, 128))

    hid = W1.shape[1]
    return pl.pallas_call(
        body,
        out_shape=[
            jax.ShapeDtypeStruct((npad, hid), jnp.float32),
            jax.ShapeDtypeStruct((npad, 1), jnp.float32),
            jax.ShapeDtypeStruct((3, 128), jnp.float32),
        ],
    )(x_pad, W1, b1_2d, beta_2d)


def _output_stage(acc, den, W2, b2_2d):
    def body(acc_ref, den_ref, w_ref, b_ref, out_ref):
        A = acc_ref[0] + acc_ref[1]
        d = den_ref[0] + den_ref[1]
        h = A / jnp.maximum(d, 1e-30)
        logits = jnp.dot(h, w_ref[...], preferred_element_type=jnp.float32)
        logits = logits + b_ref[...]
        m = jnp.max(logits, axis=1, keepdims=True)
        e = jnp.exp(logits - m)
        out_ref[...] = e / jnp.sum(e, axis=1, keepdims=True)

    npad = acc.shape[1]
    ncls = W2.shape[1]
    return pl.pallas_call(
        body,
        out_shape=jax.ShapeDtypeStruct((npad, ncls), jnp.float32),
    )(acc, den.reshape(NC, npad, 1), W2, b2_2d)


def _rsqrt16(x):
    i = plsc.bitcast(x, jnp.int32)
    i = jnp.full((L,), 0x5F3759DF, jnp.int32) - lax.shift_right_logical(
        i, jnp.full((L,), 1, jnp.int32))
    y = plsc.bitcast(i, jnp.float32)
    for _ in range(3):
        y = y * (1.5 - 0.5 * x * y * y)
    return y


def _agnn_sparse_pass(a1, a2, beta_row, srcm, dstm, first):
    npad, hid = a1.shape[-2], a1.shape[-1]
    erows = srcm.shape[0]
    rows_per_tile = erows // NW
    chunks = rows_per_tile // GROWS
    pairs = chunks // 2
    stripe = npad // NS

    mesh = plsc.VectorSubcoreMesh(core_axis_name="c", subcore_axis_name="s")

    cp = pltpu.CompilerParams()
    for fld, val in (("needs_layout_passes", False),
                     ("use_tc_tiling_on_sc", False)):
        if fld in pltpu.CompilerParams.__dataclass_fields__:
            cp = dataclasses.replace(cp, **{fld: val})

    @functools.partial(
        pl.kernel,
        compiler_params=cp,
        out_type=[
            jax.ShapeDtypeStruct((NC, npad, hid), jnp.float32),
            jax.ShapeDtypeStruct((NC, npad), jnp.float32),
        ],
        mesh=mesh,
        scratch_types=[
            pltpu.VMEM((npad,), jnp.float32),
            pltpu.VMEM((rows_per_tile, 128), jnp.int32),
            pltpu.VMEM((rows_per_tile, 128), jnp.int32),
            pltpu.VMEM((C, hid), jnp.float32),
            pltpu.VMEM((C, hid), jnp.float32),
            pltpu.VMEM((C, hid), jnp.float32),
            pltpu.VMEM((C,), jnp.float32),
            pltpu.VMEM((C, hid), jnp.float32),
            pltpu.VMEM((C, hid), jnp.float32),
            pltpu.VMEM((C, hid), jnp.float32),
            pltpu.VMEM((C,), jnp.float32),
            pltpu.VMEM((1, 128), jnp.float32),
            pltpu.VMEM((npad // NS,), jnp.float32),
            pltpu.VMEM((npad // NS,), jnp.float32),
            pltpu.VMEM_SHARED((npad, hid), jnp.float32),
            pltpu.VMEM_SHARED((npad,), jnp.float32),
            pltpu.VMEM_SHARED((npad, hid), jnp.float32),
            pltpu.VMEM_SHARED((npad,), jnp.float32),
            pltpu.SemaphoreType.DMA,
            pltpu.SemaphoreType.DMA,
            pltpu.SemaphoreType.DMA,
            pltpu.SemaphoreType.DMA,
        ],
    )
    def k(a1_hbm, a2_hbm, beta_hbm, src_hbm, dst_hbm, acc_hbm, den_hbm,
          nrm_v, src_v, dst_v, hsA, hdA, msgA, wA, hsB, hdB, msgB, wB,
          beta_v, d0_v, d1_v, tab_sh, nrm_sh, acc_sh, den_sh,
          semgA, semgB, semsA, semsB):
        cid = lax.axis_index("c")
        sid = lax.axis_index("s")
        wid = cid * NS + sid
        my_row0 = wid * rows_per_tile
        base_row = sid * stripe
        lane = lax.iota(jnp.int32, L)

        cps = [pltpu.async_copy(beta_hbm, beta_v, semgA),
               pltpu.async_copy(src_hbm.at[pl.ds(my_row0, rows_per_tile)],
                                src_v, semgA),
               pltpu.async_copy(dst_hbm.at[pl.ds(my_row0, rows_per_tile)],
                                dst_v, semgA)]
        if first:
            cps.append(pltpu.async_copy(a2_hbm, nrm_v, semgA))
            cps.append(pltpu.async_copy(a1_hbm.at[pl.ds(base_row, stripe)],
                                        tab_sh.at[pl.ds(base_row, stripe)],
                                        semgA))
            for cp in cps:
                cp.wait()
        else:
            cps.append(pltpu.async_copy(
                a2_hbm.at[0, pl.ds(base_row, stripe)], d0_v, semgA))
            cps.append(pltpu.async_copy(
                a2_hbm.at[1, pl.ds(base_row, stripe)], d1_v, semgA))
            cps.append(pltpu.async_copy(
                a1_hbm.at[0, pl.ds(base_row, stripe)],
                hsA.at[pl.ds(0, stripe)], semgA))
            cps.append(pltpu.async_copy(
                a1_hbm.at[1, pl.ds(base_row, stripe)],
                hsB.at[pl.ds(0, stripe)], semgA))
            for cp in cps:
                cp.wait()

            @pl.loop(0, stripe // L)
            def _(q):
                rows = q * L + lane
                nrm2 = jnp.zeros((L,), jnp.float32)
                a_list = []
                col_list = []
                for s in range(hid):
                    col = jnp.bitwise_and(lane + s, hid - 1)
                    a = (plsc.load_gather(hsA, [rows, col]) +
                         plsc.load_gather(hsB, [rows, col]))
                    a_list.append(a)
                    col_list.append(col)
                    nrm2 = nrm2 + a * a
                y = _rsqrt16(jnp.maximum(nrm2, 1e-24))
                d16 = d0_v[pl.ds(q * L, L)] + d1_v[pl.ds(q * L, L)]
                wA[pl.ds(q * L, L)] = (nrm2 * y) / jnp.maximum(d16, 1e-30)
                for s in range(hid):
                    plsc.store_scatter(msgA, [rows, col_list[s]],
                                       a_list[s] * y)

            pltpu.sync_copy(msgA.at[pl.ds(0, stripe)],
                            tab_sh.at[pl.ds(base_row, stripe)])
            pltpu.sync_copy(wA.at[pl.ds(0, stripe)],
                            nrm_sh.at[pl.ds(base_row, stripe)])

        @pl.loop(0, stripe)
        def _(r):
            msgA[r, :] = jnp.zeros((L,), jnp.float32)

        @pl.loop(0, stripe, step=L)
        def _(i):
            wA[pl.ds(i, L)] = jnp.zeros((L,), jnp.float32)

        pltpu.sync_copy(msgA.at[pl.ds(0, stripe)],
                        acc_sh.at[pl.ds(base_row, stripe)])
        pltpu.sync_copy(wA.at[pl.ds(0, stripe)],
                        den_sh.at[pl.ds(base_row, stripe)])
        plsc.subcore_barrier()
        if not first:
            pltpu.sync_copy(nrm_sh, nrm_v)

        b16 = beta_v[0, pl.ds(0, L)]

        def issue_gathers(t, hs, hd, semg):
            for g in range(GROWS):
                pltpu.async_copy(tab_sh.at[src_v.at[t * GROWS + g]],
                                 hs.at[pl.ds(g * 128, 128)], semg)
                pltpu.async_copy(tab_sh.at[dst_v.at[t * GROWS + g]],
                                 hd.at[pl.ds(g * 128, 128)], semg)

        def wait_gathers(hs, hd, semg):
            pltpu.make_async_copy(acc_hbm.at[0, pl.ds(0, C)], hs, semg).wait()
            pltpu.make_async_copy(acc_hbm.at[0, pl.ds(0, C)], hd, semg).wait()

        def issue_scatters(t, msg, wv, sems):
            for g in range(GROWS):
                pltpu.async_copy(msg.at[pl.ds(g * 128, 128)],
                                 acc_sh.at[dst_v.at[t * GROWS + g]],
                                 sems, add=True)
                pltpu.async_copy(wv.at[pl.ds(g * 128, 128)],
                                 den_sh.at[dst_v.at[t * GROWS + g]],
                                 sems, add=True)

        def wait_scatters(msg, wv, sems):
            pltpu.make_async_copy(acc_hbm.at[0, pl.ds(0, C)], msg, sems).wait()
            pltpu.make_async_copy(den_hbm.at[0, pl.ds(0, C)], wv, sems).wait()

        def compute(t, hs, hd, msg, wv):
            @pl.loop(0, C // L)
            def _(q):
                r0 = q * L
                rows = r0 + lane
                acc = jnp.zeros((L,), jnp.float32)
                a_list = []
                col_list = []
                for s in range(hid):
                    col = jnp.bitwise_and(lane + s, hid - 1)
                    a = plsc.load_gather(hs, [rows, col])
                    b = plsc.load_gather(hd, [rows, col])
                    a_list.append(a)
                    col_list.append(col)
                    acc = acc + a * b
                w16 = jnp.exp(acc * b16)
                lrow = t * GROWS + q // (128 // L)
                c0 = (q % (128 // L)) * L
                src16 = src_v[lrow, pl.ds(c0, L)]
                ns16 = plsc.load_gather(nrm_v, [src16])
                v16 = w16 * ns16
                wv[pl.ds(r0, L)] = w16
                for s in range(hid):
                    plsc.store_scatter(msg, [rows, col_list[s]],
                                       v16 * a_list[s])

        issue_gathers(0, hsA, hdA, semgA)
        issue_gathers(1, hsB, hdB, semgB)

        @pl.loop(0, pairs)
        def _(tt):
            t0 = 2 * tt
            t1 = t0 + 1

            wait_gathers(hsA, hdA, semgA)

            @pl.when(tt > 0)
            def _():
                wait_scatters(msgA, wA, semsA)

            compute(t0, hsA, hdA, msgA, wA)
            issue_scatters(t0, msgA, wA, semsA)

            @pl.when(tt < pairs - 1)
            def _():
                issue_gathers(t0 + 2, hsA, hdA, semgA)

            wait_gathers(hsB, hdB, semgB)

            @pl.when(tt > 0)
            def _():
                wait_scatters(msgB, wB, semsB)

            compute(t1, hsB, hdB, msgB, wB)
            issue_scatters(t1, msgB, wB, semsB)

            @pl.when(tt < pairs - 1)
            def _():
                issue_gathers(t1 + 2, hsB, hdB, semgB)

        wait_scatters(msgA, wA, semsA)
        wait_scatters(msgB, wB, semsB)

        plsc.subcore_barrier()
        pltpu.sync_copy(acc_sh.at[pl.ds(base_row, stripe)],
                        acc_hbm.at[cid, pl.ds(base_row, stripe)])
        pltpu.sync_copy(den_sh.at[pl.ds(base_row, stripe)],
                        den_hbm.at[cid, pl.ds(base_row, stripe)])

    return k(a1, a2, beta_row, srcm, dstm)


def kernel(x, edge_index, W1, b1, beta1, beta2, beta3, W2, b2):
    n, nfeat = x.shape
    e = edge_index.shape[1]
    hid = W1.shape[1]

    npad = ((n + 16) + NW * L - 1) // (NW * L) * (NW * L)
    etot = e + n
    epad = (etot + 2 * NW * C - 1) // (2 * NW * C) * (2 * NW * C)
    erows = epad // 128

    loop_idx = jnp.arange(n, dtype=jnp.int32)
    pad_idx = jnp.arange(epad - etot, dtype=jnp.int32)
    src = jnp.concatenate([edge_index[0].astype(jnp.int32), loop_idx,
                           pad_idx % n])
    dst = jnp.concatenate([edge_index[1].astype(jnp.int32), loop_idx,
                           n + pad_idx % (npad - n)])
    srcm = src.reshape(erows, 128)
    dstm = dst.reshape(erows, 128)

    x_pad = jnp.pad(x, ((0, npad - n), (0, 0)))
    b1_2d = b1.reshape(1, hid)
    betas31 = jnp.stack([beta1, beta2, beta3]).astype(jnp.float32).reshape(3, 1)

    hn, nrm2, beta_rows = _input_stage(x_pad, W1, b1_2d, betas31, n, npad)
    acc, den = _agnn_sparse_pass(hn, nrm2.reshape(npad), beta_rows[0:1],
                                 srcm, dstm, True)
    acc, den = _agnn_sparse_pass(acc, den, beta_rows[1:2], srcm, dstm, False)
    acc, den = _agnn_sparse_pass(acc, den, beta_rows[2:3], srcm, dstm, False)

    out = _output_stage(acc, den, W2, b2.reshape(1, -1))
    return out[:n]
